# baseline (speedup 1.0000x reference)
import functools

import jax
import jax.numpy as jnp
from jax import lax
from jax.experimental import pallas as pl
from jax.experimental.pallas import tpu as pltpu
from jax.experimental.pallas import tpu_sc as plsc

VOCAB = 1000
BATCH = 1024
SEQ = 50
NC = 2
NS = 16
NW = NC * NS
BPW = BATCH // NW
RPW = BPW * SEQ
RPAD = RPW + 64

_mesh = plsc.VectorSubcoreMesh(core_axis_name="c", subcore_axis_name="s")


@functools.partial(
    pl.kernel,
    mesh=_mesh,
    out_type=jax.ShapeDtypeStruct((BATCH, SEQ, VOCAB), jnp.float32),
    compiler_params=pltpu.CompilerParams(
        needs_layout_passes=False, use_tc_tiling_on_sc=False
    ),
    scratch_types=[
        pltpu.VMEM((RPAD,), jnp.int32),
        pltpu.VMEM((RPAD,), jnp.float32),
        pltpu.VMEM((SEQ, VOCAB), jnp.float32),
        pltpu.VMEM((SEQ, VOCAB), jnp.float32),
        pltpu.SemaphoreType.DMA,
        pltpu.SemaphoreType.DMA,
    ],
)
def _onehot_sc(idx_hbm, msk_hbm, zeros_hbm, out_hbm, idx_v, msk_v, buf0, buf1, sem0, sem1):
    wid = lax.axis_index("s") * NC + lax.axis_index("c")
    b0 = wid * BPW

    pltpu.sync_copy(idx_hbm.at[pl.ds(b0 * SEQ, RPW)], idx_v.at[pl.ds(0, RPW)])
    pltpu.sync_copy(msk_hbm.at[pl.ds(b0 * SEQ, RPW)], msk_v.at[pl.ds(0, RPW)])

    zeros16 = jnp.zeros((16,), jnp.float32)

    pltpu.sync_copy(zeros_hbm, buf0)
    pltpu.sync_copy(zeros_hbm, buf1)

    lane = lax.iota(jnp.int32, 16)
    bufs = (buf0, buf1)
    sems = (sem0, sem1)

    def scatter_chunk(buf, chunk, values16):
        for k in range(4):
            t = lane + k * 16
            iv = idx_v[pl.ds(chunk * SEQ + k * 16, 16)]
            mv = values16(k)
            if k < 3:
                plsc.store_scatter(buf, [t, iv], mv)
            else:
                plsc.store_scatter(buf, [t, iv], mv, mask=t < SEQ)

    def cbody(i, carry):
        for b in range(2):
            chunk = i * 2 + b
            buf = bufs[b]
            sem = sems[b]

            @pl.when(chunk >= 2)
            def _():
                prev = chunk - 2
                pltpu.make_async_copy(
                    buf, out_hbm.at[b0 + prev], sem
                ).wait()
                scatter_chunk(buf, prev, lambda k: zeros16)

            scatter_chunk(
                buf, chunk,
                lambda k: msk_v[pl.ds(chunk * SEQ + k * 16, 16)],
            )
            pltpu.make_async_copy(buf, out_hbm.at[b0 + chunk], sem).start()
        return carry

    lax.fori_loop(0, BPW // 2, cbody, 0)

    for b in range(2):
        chunk = BPW - 2 + b
        pltpu.make_async_copy(bufs[b], out_hbm.at[b0 + chunk], sems[b]).wait()


def kernel(array, mask):
    idx = array.reshape(BATCH * SEQ).astype(jnp.int32)
    msk = mask.reshape(BATCH * SEQ).astype(jnp.float32)
    zeros = jnp.zeros((SEQ, VOCAB), jnp.float32)
    return _onehot_sc(idx, msk, zeros)


# speedup vs baseline: 4.2069x; 4.2069x over previous
import functools

import jax
import jax.numpy as jnp
from jax import lax
from jax.experimental import pallas as pl
from jax.experimental.pallas import tpu as pltpu
from jax.experimental.pallas import tpu_sc as plsc

VOCAB = 1000
BATCH = 1024
SEQ = 50
N = BATCH * SEQ * VOCAB
NC = 2
NS = 16
NW = NC * NS
EPW = N // NW
ZCH = 64000
NFILL = EPW // ZCH
SLABS_PER_SC = SEQ // NC
SLAB = VOCAB * BATCH

_mesh = plsc.VectorSubcoreMesh(core_axis_name="c", subcore_axis_name="s")


@functools.partial(
    pl.kernel,
    mesh=_mesh,
    out_type=jax.ShapeDtypeStruct((N,), jnp.float32),
    compiler_params=pltpu.CompilerParams(
        needs_layout_passes=False, use_tc_tiling_on_sc=False
    ),
    scratch_types=[
        pltpu.VMEM((ZCH,), jnp.float32),
        pltpu.VMEM((BATCH,), jnp.int32),
        pltpu.VMEM((BATCH,), jnp.float32),
        pltpu.VMEM((8, 128), jnp.int32),
        pltpu.VMEM((8, 128), jnp.float32),
        pltpu.SemaphoreType.DMA,
        pltpu.SemaphoreType.DMA,
    ],
)
def _onehot_sc(idxT_hbm, mskT_hbm, zeros_hbm, out_hbm,
               zbuf, sidx, smsk, abuf, vbuf, fsem, ssem):
    cid = lax.axis_index("c")
    sid = lax.axis_index("s")
    wid = cid * NS + sid
    base = wid * EPW

    pltpu.sync_copy(zeros_hbm, zbuf)

    def fstart(i, carry):
        pltpu.make_async_copy(
            zbuf, out_hbm.at[pl.ds(base + i * ZCH, ZCH)], fsem
        ).start()
        return carry

    lax.fori_loop(0, NFILL, fstart, 0)

    def fwait(i, carry):
        pltpu.make_async_copy(
            zbuf, out_hbm.at[pl.ds(base + i * ZCH, ZCH)], fsem
        ).wait()
        return carry

    lax.fori_loop(0, NFILL, fwait, 0)

    plsc.subcore_barrier()

    lane = lax.iota(jnp.int32, 16)

    def do_slab(sub, carry):
        @pl.when(sub * NS // SLABS_PER_SC == sid)
        def _():
            tg = cid * SLABS_PER_SC + sub
            pltpu.sync_copy(idxT_hbm.at[pl.ds(tg * BATCH, BATCH)], sidx)
            pltpu.sync_copy(mskT_hbm.at[pl.ds(tg * BATCH, BATCH)], smsk)
            tbase = tg * SLAB
            for k in range(BATCH // 16):
                iv = sidx[pl.ds(k * 16, 16)]
                mv = smsk[pl.ds(k * 16, 16)]
                va = ((iv >> 3) << 13) + ((iv & 7) << 7)
                addr = va + (tbase + (k // 8) * 1024 + (k % 8) * 16) + lane
                row = jnp.full((16,), k // 8, jnp.int32)
                col = (k % 8) * 16 + lane
                plsc.store_scatter(abuf, [row, col], addr)
                plsc.store_scatter(vbuf, [row, col], mv)
            for j in range(8):
                pltpu.make_async_copy(
                    vbuf.at[j], out_hbm.at[abuf.at[j]], ssem
                ).start()
            for j in range(8):
                pltpu.make_async_copy(
                    vbuf.at[j], out_hbm.at[abuf.at[j]], ssem
                ).wait()
        return carry

    lax.fori_loop(0, SLABS_PER_SC, do_slab, 0)


def kernel(array, mask):
    idxT = array.astype(jnp.int32).T.reshape(SEQ * BATCH)
    mskT = mask.astype(jnp.float32).T.reshape(SEQ * BATCH)
    zeros = jnp.zeros((ZCH,), jnp.float32)
    out = _onehot_sc(idxT, mskT, zeros)
    out5 = out.reshape(SEQ, VOCAB // 8, 8, 8, 128)
    return out5.transpose(2, 4, 0, 1, 3).reshape(BATCH, SEQ, VOCAB)
